# Initial kernel scaffold; baseline (speedup 1.0000x reference)
import jax
import jax.numpy as jnp
from jax.experimental import pallas as pl


def kernel(x, epoch, emb, W, b):
    raise NotImplementedError("write your pallas kernel here")


# speedup vs baseline: 63.3093x; 63.3093x over previous
import functools

import jax
import jax.numpy as jnp
import numpy as np
from jax import lax
from jax.experimental import pallas as pl
from jax.experimental.pallas import tpu as pltpu
from jax.experimental.pallas import tpu_sc as plsc

_T = 2048
_MASK_IDX = 1
_ALPHA = 1.5
_WARMUP = 10
_VOCAB = 32
_D = 1024
_B = 128
_S = 2048

_NC = 2
_NS = 16
_NW = _NC * _NS
_ROWS = _B // _NW
_LANES = 16


@functools.lru_cache(maxsize=1)
def _rank_table() -> np.ndarray:
    with jax.ensure_compile_time_eval():
        scores = np.asarray(jax.random.uniform(jax.random.key(2), (_B, _S)))
    order = np.argsort(-scores, axis=1, kind="stable")
    ranks = np.empty((_B, _S), dtype=np.int32)
    rows = np.arange(_B)[:, None]
    ranks[rows, order] = np.arange(_S, dtype=np.int32)[None, :]
    return ranks


def _g_table_body(emb_ref, w_ref, b_ref, t_ref, n_ref, out_ref):
    w = w_ref[...]
    e1 = emb_ref[pl.ds(_MASK_IDX, 1), :]
    r1 = jnp.dot(e1, w, preferred_element_type=jnp.float32)
    csw = jnp.sum(w, axis=0, keepdims=True)
    t_f = t_ref[...]
    logits = r1 + b_ref[...] + (t_f * (1.0 / _T)) * csw
    mx = jnp.max(logits, axis=1, keepdims=True)
    lse = mx + jnp.log(jnp.sum(jnp.exp(logits - mx), axis=1, keepdims=True))
    g = lse - logits
    n_f = n_ref[...]
    denom = jnp.maximum(jnp.sum(n_f), 1.0)
    ratios = n_f * (1.0 / _S)
    wmean = jnp.mean(ratios * jnp.sqrt(ratios))
    out_ref[...] = g * (wmean / denom)


def _g_table(emb, w, b, t_f, n_f):
    return pl.pallas_call(
        _g_table_body,
        out_shape=jax.ShapeDtypeStruct((_B, _VOCAB), jnp.float32),
    )(emb, w, b.reshape(1, _VOCAB), t_f.reshape(_B, 1), n_f.reshape(_B, 1))


def _sc_body(x_hbm, rk_hbm, n_hbm, g_hbm, out_hbm, xv, rv, nv, gv, av):
    c = lax.axis_index("c")
    s = lax.axis_index("s")
    wid = s * _NC + c
    base = wid * _ROWS
    pltpu.sync_copy(x_hbm.at[pl.ds(base, _ROWS)], xv)
    pltpu.sync_copy(rk_hbm.at[pl.ds(base, _ROWS)], rv)
    pltpu.sync_copy(n_hbm, nv)
    pltpu.sync_copy(g_hbm, gv)
    acc = jnp.zeros((_LANES,), jnp.float32)
    for r in range(_ROWS):
        row_splat = jnp.full((_LANES,), base + r, jnp.int32)
        n_splat = plsc.load_gather(nv, [row_splat])

        def chunk(i, a, r=r, row_splat=row_splat, n_splat=n_splat):
            xvv = xv[r, pl.ds(i * _LANES, _LANES)]
            rvv = rv[r, pl.ds(i * _LANES, _LANES)]
            g = plsc.load_gather(gv, [row_splat, xvv])
            return a + jnp.where(rvv < n_splat, g, 0.0)

        acc = lax.fori_loop(0, _S // _LANES, chunk, acc)
    av[...] = acc
    pltpu.sync_copy(av, out_hbm.at[wid])


def _sc_sweep(x, ranks, n_i, gs):
    mesh = plsc.VectorSubcoreMesh(core_axis_name="c", subcore_axis_name="s")
    return pl.kernel(
        _sc_body,
        out_type=jax.ShapeDtypeStruct((_NW, _LANES), jnp.float32),
        mesh=mesh,
        compiler_params=pltpu.CompilerParams(needs_layout_passes=False),
        scratch_types=[
            pltpu.VMEM((_ROWS, _S), jnp.int32),
            pltpu.VMEM((_ROWS, _S), jnp.int32),
            pltpu.VMEM((_B,), jnp.int32),
            pltpu.VMEM((_B, _VOCAB), jnp.float32),
            pltpu.VMEM((_LANES,), jnp.float32),
        ],
    )(x, ranks, n_i, gs)


def kernel(x, epoch, emb, W, b):
    progress = jnp.minimum(epoch / max(_WARMUP, 1), 1.0)
    t_ceiling = jnp.clip(
        jnp.floor(1 + (_T - 1) * progress).astype(jnp.int32), 1, _T)
    t = jax.random.randint(jax.random.key(1), (_B,), 1, t_ceiling + 1)
    t_f = t.astype(jnp.float32)
    n_i = jnp.clip(
        jnp.ceil(t_f * (float(_S) / _T)).astype(jnp.int32), 1, _S)
    n_f = n_i.astype(jnp.float32)
    ranks = jnp.asarray(_rank_table())

    gs = _g_table(emb, W, b, t_f, n_f)
    partials = _sc_sweep(x, ranks, n_i, gs)
    return jnp.sum(partials)


# speedup vs baseline: 75.1542x; 1.1871x over previous
import functools

import jax
import jax.numpy as jnp
import numpy as np
from jax import lax
from jax.experimental import pallas as pl
from jax.experimental.pallas import tpu as pltpu
from jax.experimental.pallas import tpu_sc as plsc

_T = 2048
_MASK_IDX = 1
_ALPHA = 1.5
_WARMUP = 10
_VOCAB = 32
_D = 1024
_B = 128
_S = 2048

_NC = 2
_NS = 16
_NW = _NC * _NS
_ROWS = _B // _NW
_LANES = 16


@functools.lru_cache(maxsize=1)
def _consts():
    with jax.ensure_compile_time_eval():
        scores = np.asarray(jax.random.uniform(jax.random.key(2), (_B, _S)))
        k1, k2 = jax.random.split(jax.random.key(1))
        hi = np.asarray(jax.random.bits(k1, (_B,), np.uint32))
        lo = np.asarray(jax.random.bits(k2, (_B,), np.uint32))
    order = np.argsort(-scores, axis=1, kind="stable")
    ranks = np.empty((_B, _S), dtype=np.int32)
    rows = np.arange(_B)[:, None]
    ranks[rows, order] = np.arange(_S, dtype=np.int32)[None, :]
    return ranks.reshape(-1), hi.reshape(_B, 1), lo.reshape(_B, 1)


def _prep_body(ep_ref, hi_ref, lo_ref, emb_ref, w_ref, b_ref, gs_ref, n_ref):
    epf = ep_ref[...].astype(jnp.float32)
    progress = jnp.minimum(epf * (1.0 / _WARMUP), 1.0)
    tceil = jnp.clip(
        jnp.floor(1.0 + (_T - 1) * progress).astype(jnp.int32), 1, _T)
    span = tceil.astype(jnp.uint32)
    mult = jnp.uint32(2 ** 16) % span
    mult = (mult * mult) % span
    off = (hi_ref[...] % span) * mult + (lo_ref[...] % span)
    t = (off % span).astype(jnp.int32) + 1
    t_f = t.astype(jnp.float32)
    n = jnp.clip(
        jnp.ceil(t_f * (float(_S) / _T)).astype(jnp.int32), 1, _S)
    n_f = n.astype(jnp.float32)
    w = w_ref[...]
    e1 = emb_ref[pl.ds(_MASK_IDX, 1), :]
    r1 = jnp.dot(e1, w, preferred_element_type=jnp.float32)
    csw = jnp.sum(w, axis=0, keepdims=True)
    logits = r1 + b_ref[...] + (t_f * (1.0 / _T)) * csw
    mx = jnp.max(logits, axis=1, keepdims=True)
    lse = mx + jnp.log(jnp.sum(jnp.exp(logits - mx), axis=1, keepdims=True))
    g = lse - logits
    denom = jnp.maximum(jnp.sum(n_f), 1.0)
    ratios = n_f * (1.0 / _S)
    wmean = jnp.mean(ratios * jnp.sqrt(ratios))
    gs_ref[...] = g * (wmean / denom)
    n_ref[...] = n


def _prep(ep, hi, lo, emb, w, b):
    return pl.pallas_call(
        _prep_body,
        out_shape=(
            jax.ShapeDtypeStruct((_B, _VOCAB), jnp.float32),
            jax.ShapeDtypeStruct((_B, 1), jnp.int32),
        ),
    )(ep, hi, lo, emb, w, b)


def _sc_body(x_hbm, rk_hbm, n_hbm, g_hbm, out_hbm, xv, rv, nv, gv, av):
    c = lax.axis_index("c")
    s = lax.axis_index("s")
    wid = s * _NC + c
    base = wid * _ROWS
    pltpu.sync_copy(x_hbm.at[pl.ds(base, _ROWS)], xv)
    pltpu.sync_copy(rk_hbm.at[pl.ds(base * _S, _ROWS * _S)], rv)
    pltpu.sync_copy(n_hbm, nv)
    pltpu.sync_copy(g_hbm, gv)
    zeros16 = jnp.zeros((_LANES,), jnp.int32)
    acc = jnp.zeros((_LANES,), jnp.float32)
    for r in range(_ROWS):
        row_splat = jnp.full((_LANES,), base + r, jnp.int32)
        n_splat = plsc.load_gather(nv, [row_splat, zeros16])

        def chunk(i, a, r=r, row_splat=row_splat, n_splat=n_splat):
            xvv = xv[r, pl.ds(i * _LANES, _LANES)]
            rvv = rv[pl.ds(r * _S + i * _LANES, _LANES)]
            g = plsc.load_gather(gv, [row_splat, xvv])
            return a + jnp.where(rvv < n_splat, g, 0.0)

        acc = plsc.parallel_loop(0, _S // _LANES, unroll=8, carry=acc)(chunk)
    av[...] = acc
    pltpu.sync_copy(av, out_hbm.at[wid])


def _sc_sweep(x, ranks_flat, n_i, gs):
    mesh = plsc.VectorSubcoreMesh(core_axis_name="c", subcore_axis_name="s")
    return pl.kernel(
        _sc_body,
        out_type=jax.ShapeDtypeStruct((_NW, _LANES), jnp.float32),
        mesh=mesh,
        compiler_params=pltpu.CompilerParams(needs_layout_passes=False),
        scratch_types=[
            pltpu.VMEM((_ROWS, _S), jnp.int32),
            pltpu.VMEM((_ROWS * _S,), jnp.int32),
            pltpu.VMEM((_B, 1), jnp.int32),
            pltpu.VMEM((_B, _VOCAB), jnp.float32),
            pltpu.VMEM((_LANES,), jnp.float32),
        ],
    )(x, ranks_flat, n_i, gs)


def kernel(x, epoch, emb, W, b):
    ranks_flat, hi, lo = _consts()
    ep = jnp.asarray(epoch, jnp.int32).reshape(1, 1)
    gs, n_i = _prep(ep, jnp.asarray(hi), jnp.asarray(lo),
                    emb, W, b.reshape(1, _VOCAB))
    partials = _sc_sweep(x, jnp.asarray(ranks_flat), n_i, gs)
    return jnp.sum(partials)
